# Initial kernel scaffold; baseline (speedup 1.0000x reference)
import jax
import jax.numpy as jnp
from jax.experimental import pallas as pl


def kernel(z, codebook):
    raise NotImplementedError("write your pallas kernel here")


# speedup vs baseline: 1.4942x; 1.4942x over previous
import functools

import jax
import jax.numpy as jnp
from jax import lax
from jax.experimental import pallas as pl
from jax.experimental.pallas import tpu as pltpu
from jax.experimental.pallas import tpu_sc as plsc

_EPS = 1e-12
_BETA = 0.25


def _cbnorm_body(cb_ref, en_ref, c_ref):
    cb = cb_ref[...]
    n = jnp.sqrt(jnp.sum(cb * cb, axis=1, keepdims=True))
    en = cb / jnp.maximum(n, _EPS)
    en_ref[...] = en
    c_ref[...] = jnp.sum(en * en, axis=1, keepdims=True).reshape(1, -1)


def _normalize_codebook(codebook, block):
    n_e, e_dim = codebook.shape
    grid = (n_e // block,)
    return pl.pallas_call(
        _cbnorm_body,
        grid=grid,
        in_specs=[pl.BlockSpec((block, e_dim), lambda j: (j, 0))],
        out_specs=[
            pl.BlockSpec((block, e_dim), lambda j: (j, 0)),
            pl.BlockSpec((1, block), lambda j: (0, j)),
        ],
        out_shape=[
            jax.ShapeDtypeStruct((n_e, e_dim), jnp.float32),
            jax.ShapeDtypeStruct((1, n_e), jnp.float32),
        ],
    )(codebook)


def _argmin_body(n_e, n_win, zf_ref, ent_ref, c_ref, idx_ref):
    z = zf_ref[...]
    n = jnp.sqrt(jnp.sum(z * z, axis=1, keepdims=True))
    zn = z / jnp.maximum(n, _EPS)
    a = jnp.sum(zn * zn, axis=1, keepdims=True)
    s = lax.dot_general(zn, ent_ref[...], (((1,), (0,)), ((), ())),
                        preferred_element_type=jnp.float32)
    d = (a + c_ref[...]) - 2.0 * s
    w_size = n_e // n_win
    best_i = None
    best_v = None
    for w in range(n_win):
        dw = d[:, w * w_size:(w + 1) * w_size]
        lmin = jnp.min(dw, axis=1, keepdims=True)
        col = lax.broadcasted_iota(jnp.int32, dw.shape, 1)
        larg = jnp.min(jnp.where(dw == lmin, col, w_size), axis=1,
                       keepdims=True) + w * w_size
        if w == 0:
            best_i = larg
            best_v = lmin.astype(jnp.bfloat16)
        else:
            upd = lmin < best_v.astype(jnp.float32)
            best_i = jnp.where(upd, larg, best_i)
            best_v = jnp.where(upd, lmin, best_v.astype(jnp.float32)
                               ).astype(jnp.bfloat16)
    idx_ref[...] = best_i


def _argmin_indices(z_flat, emb_n_t, c, br):
    b, e_dim = z_flat.shape
    n_e = emb_n_t.shape[1]
    grid = (b // br,)
    return pl.pallas_call(
        functools.partial(_argmin_body, n_e, 2),
        grid=grid,
        in_specs=[
            pl.BlockSpec((br, e_dim), lambda i: (i, 0)),
            pl.BlockSpec((e_dim, n_e), lambda i: (0, 0)),
            pl.BlockSpec((1, n_e), lambda i: (0, 0)),
        ],
        out_specs=pl.BlockSpec((br, 1), lambda i: (i, 0)),
        out_shape=jax.ShapeDtypeStruct((b, 1), jnp.int32),
        compiler_params=pltpu.CompilerParams(
            dimension_semantics=("arbitrary",),
            vmem_limit_bytes=100663296),
    )(z_flat, emb_n_t, c)


def _sc_gather_counts(codebook, idx2d, n_e, b):
    mesh = plsc.VectorSubcoreMesh(core_axis_name="c", subcore_axis_name="s")
    e_dim = codebook.shape[1]
    n_chunks = b // (32 * 128)
    bins_per_sub = n_e // 16

    @functools.partial(
        pl.kernel,
        out_type=[
            jax.ShapeDtypeStruct((b, e_dim), jnp.float32),
            jax.ShapeDtypeStruct((2, n_e), jnp.float32),
        ],
        mesh=mesh,
        scratch_types=[
            pltpu.VMEM((n_chunks, 128), jnp.int32),
            pltpu.VMEM((128, e_dim), jnp.float32),
            pltpu.VMEM((128,), jnp.float32),
            pltpu.VMEM((bins_per_sub,), jnp.float32),
            pltpu.VMEM_SHARED((n_e,), jnp.float32),
        ],
    )
    def k(cb_hbm, idx_hbm, zq_hbm, cnt_hbm, idx_v, rows_v, ones_v, zero_v, hist_sh):
        cid = lax.axis_index("c")
        sid = lax.axis_index("s")
        wid = sid * 2 + cid
        base_row = wid * n_chunks

        for m in range(bins_per_sub // 16):
            zero_v[pl.ds(m * 16, 16)] = jnp.zeros((16,), jnp.float32)
        pltpu.sync_copy(zero_v, hist_sh.at[pl.ds(sid * bins_per_sub, bins_per_sub)])
        for m in range(8):
            ones_v[pl.ds(m * 16, 16)] = jnp.ones((16,), jnp.float32)
        pltpu.sync_copy(idx_hbm.at[pl.ds(base_row, n_chunks)], idx_v)
        plsc.subcore_barrier()

        for kk in range(n_chunks):
            pltpu.sync_copy(cb_hbm.at[idx_v.at[kk]], rows_v)
            pltpu.sync_copy(rows_v, zq_hbm.at[pl.ds((base_row + kk) * 128, 128)])
            pltpu.sync_copy(ones_v, hist_sh.at[idx_v.at[kk]], add=True)

        plsc.subcore_barrier()
        pltpu.sync_copy(hist_sh.at[pl.ds(sid * bins_per_sub, bins_per_sub)],
                        zero_v)
        pltpu.sync_copy(zero_v, cnt_hbm.at[cid, pl.ds(sid * bins_per_sub, bins_per_sub)])

    return k(codebook, idx2d)


def _final_body(n_tok, e_dim, z_ref, zq_ref, cnt_ref, out_ref, loss_ref, perp_ref):
    i = pl.program_id(0)
    z = z_ref[...]
    zq = zq_ref[...]
    zn = z / jnp.maximum(jnp.sqrt(jnp.sum(z * z, axis=1, keepdims=True)), _EPS)
    zqn = zq / jnp.maximum(jnp.sqrt(jnp.sum(zq * zq, axis=1, keepdims=True)), _EPS)
    st = z + (zq - z)
    out_ref[...] = st / jnp.maximum(
        jnp.sqrt(jnp.sum(st * st, axis=1, keepdims=True)), _EPS)
    diff = zqn - zn
    part = jnp.sum(jnp.sum(diff * diff, axis=1, keepdims=True), axis=0,
                   keepdims=True)

    @pl.when(i == 0)
    def _():
        loss_ref[...] = jnp.zeros((1, 1), jnp.float32)
        cnt = cnt_ref[0:1, :] + cnt_ref[1:2, :]
        em = cnt / jnp.float32(n_tok)
        h = -jnp.sum(em * jnp.log(em + 1e-10), axis=1, keepdims=True)
        perp_ref[...] = jnp.exp(h)

    scale = (1.0 + _BETA) / (n_tok * e_dim)
    loss_ref[...] = loss_ref[...] + part * scale


def _finalize(z_flat, zq, counts, br):
    b, e_dim = z_flat.shape
    n_e = counts.shape[1]
    grid = (b // br,)
    return pl.pallas_call(
        functools.partial(_final_body, b, e_dim),
        grid=grid,
        in_specs=[
            pl.BlockSpec((br, e_dim), lambda i: (i, 0)),
            pl.BlockSpec((br, e_dim), lambda i: (i, 0)),
            pl.BlockSpec((2, n_e), lambda i: (0, 0)),
        ],
        out_specs=[
            pl.BlockSpec((br, e_dim), lambda i: (i, 0)),
            pl.BlockSpec((1, 1), lambda i: (0, 0)),
            pl.BlockSpec((1, 1), lambda i: (0, 0)),
        ],
        out_shape=[
            jax.ShapeDtypeStruct((b, e_dim), jnp.float32),
            jax.ShapeDtypeStruct((1, 1), jnp.float32),
            jax.ShapeDtypeStruct((1, 1), jnp.float32),
        ],
        compiler_params=pltpu.CompilerParams(
            dimension_semantics=("arbitrary",)),
    )(z_flat, zq, counts)


def kernel(z, codebook):
    n_e, e_dim = codebook.shape
    z_flat = z.reshape(-1, e_dim)
    b = z_flat.shape[0]

    emb_n, c = _normalize_codebook(codebook, block=1024)
    idx = _argmin_indices(z_flat, emb_n.T, c, br=512)
    idx2d = idx.reshape(b // 128, 128)
    zq, counts = _sc_gather_counts(codebook, idx2d, n_e, b)
    zq_out, loss, perp = _finalize(z_flat, zq, counts, br=2048)

    return (loss.reshape(()), zq_out.reshape(z.shape),
            idx.reshape(b), perp.reshape(()))
